# baseline (speedup 1.0000x reference)
import functools

import jax
import jax.numpy as jnp
from jax import lax
from jax.experimental import pallas as pl
from jax.experimental.pallas import tpu as pltpu
from jax.experimental.pallas import tpu_sc as plsc

F32 = jnp.float32
_N = 10000
_E = 320000
_H = 8
_D = 128
_LB = 128
_NW = 32
_NBLK = _E // _LB
_BN = 2000
_NP = 10112
_RPT = _NP // 16


def _tc_embed(x_ref, w_ref, as_ref, ad_ref, h_ref, ats_ref, atd_ref,
              ms_ref, md_ref):
    h = jnp.dot(x_ref[...], w_ref[...], preferred_element_type=F32)
    h_ref[...] = h
    a_s = jnp.dot(h, as_ref[...], preferred_element_type=F32)
    a_d = jnp.dot(h, ad_ref[...], preferred_element_type=F32)
    ats_ref[...] = a_s
    atd_ref[...] = a_d
    bs = jnp.max(a_s, axis=0, keepdims=True)
    bd = jnp.max(a_d, axis=0, keepdims=True)
    i = pl.program_id(0)

    @pl.when(i == 0)
    def _():
        ms_ref[...] = bs
        md_ref[...] = bd

    @pl.when(i > 0)
    def _():
        ms_ref[...] = jnp.maximum(ms_ref[...], bs)
        md_ref[...] = jnp.maximum(md_ref[...], bd)


def _embed_call(x, W, As, Ad):
    return pl.pallas_call(
        _tc_embed,
        grid=(_N // _BN,),
        in_specs=[
            pl.BlockSpec((_BN, _D), lambda i: (i, 0)),
            pl.BlockSpec((_D, _D), lambda i: (0, 0)),
            pl.BlockSpec((_D, 16), lambda i: (0, 0)),
            pl.BlockSpec((_D, 16), lambda i: (0, 0)),
        ],
        out_specs=[
            pl.BlockSpec((_BN, _D), lambda i: (i, 0)),
            pl.BlockSpec((_BN, 16), lambda i: (i, 0)),
            pl.BlockSpec((_BN, 16), lambda i: (i, 0)),
            pl.BlockSpec((1, 16), lambda i: (0, 0)),
            pl.BlockSpec((1, 16), lambda i: (0, 0)),
        ],
        out_shape=[
            jax.ShapeDtypeStruct((_N, _D), F32),
            jax.ShapeDtypeStruct((_N, 16), F32),
            jax.ShapeDtypeStruct((_N, 16), F32),
            jax.ShapeDtypeStruct((1, 16), F32),
            jax.ShapeDtypeStruct((1, 16), F32),
        ],
    )(x, W, As, Ad)


def _sc_edge_body(src_hbm, dst_hbm, ats_hbm, atd_hbm, h_hbm, ms_hbm, md_hbm,
                  dpart, opart,
                  sidx, didx, asr, adr, hr, exr, mv, zb, zbd, dsp, osp,
                  sem_a, sem_b, sem_c):
    cid = lax.axis_index("c")
    sid = lax.axis_index("s")
    wid = sid * 2 + cid

    pltpu.sync_copy(ms_hbm, mv)
    m_s = mv[...]
    pltpu.sync_copy(md_hbm, mv)
    msum = m_s + mv[...]
    mshift = jnp.where(msum > 0, msum, 0.2 * msum)

    z16 = jnp.zeros((16,), F32)
    for r in range(8):
        for k in range(_D // 16):
            zb[r, pl.ds(16 * k, 16)] = z16
        zbd[r, pl.ds(0, 16)] = z16
    rbase = pl.multiple_of(sid * _RPT, 8)

    def zero_blk(r, c0):
        off = pl.multiple_of(rbase + r * 8, 8)
        pltpu.sync_copy(zb, osp.at[pl.ds(off, 8)])
        pltpu.sync_copy(zbd, dsp.at[pl.ds(off, 8)])
        return c0

    lax.fori_loop(0, _RPT // 8, zero_blk, 0)
    plsc.subcore_barrier()

    nb = (_NBLK + _NW - 1 - wid) // _NW

    def blk(j, carry):
        base = (wid + j * _NW) * _LB
        pltpu.sync_copy(src_hbm.at[pl.ds(base, _LB)], sidx)
        pltpu.sync_copy(dst_hbm.at[pl.ds(base, _LB)], didx)
        ca = pltpu.async_copy(ats_hbm.at[sidx], asr, sem_a)
        cb = pltpu.async_copy(atd_hbm.at[didx], adr, sem_b)
        cc = pltpu.async_copy(h_hbm.at[sidx], hr, sem_c)
        ca.wait()
        cb.wait()
        cc.wait()

        def edge(i, c2):
            al = asr[i, :] + adr[i, :]
            al = jnp.where(al > 0, al, 0.2 * al)
            e = jnp.exp(al - mshift)
            exr[i, :] = e
            for hh in range(_H):
                w_hh = e[hh]
                hr[i, pl.ds(16 * hh, 16)] = hr[i, pl.ds(16 * hh, 16)] * w_hh
            return c2

        lax.fori_loop(0, _LB, edge, 0, unroll=4)

        pltpu.sync_copy(exr, dsp.at[didx], add=True)
        pltpu.sync_copy(hr, osp.at[didx], add=True)
        return carry

    lax.fori_loop(0, nb, blk, 0)
    plsc.subcore_barrier()

    pltpu.sync_copy(dsp.at[pl.ds(rbase, _RPT)],
                    dpart.at[cid, pl.ds(rbase, _RPT)])
    pltpu.sync_copy(osp.at[pl.ds(rbase, _RPT)],
                    opart.at[cid, pl.ds(rbase, _RPT)])


_sc_edge = functools.partial(
    pl.kernel,
    out_type=[
        jax.ShapeDtypeStruct((2, _NP, 16), F32),
        jax.ShapeDtypeStruct((2, _NP, _D), F32),
    ],
    mesh=plsc.VectorSubcoreMesh(core_axis_name="c", subcore_axis_name="s"),
    compiler_params=pltpu.CompilerParams(use_tc_tiling_on_sc=False),
    scratch_types=[
        pltpu.VMEM((_LB,), jnp.int32),
        pltpu.VMEM((_LB,), jnp.int32),
        pltpu.VMEM((_LB, 16), F32),
        pltpu.VMEM((_LB, 16), F32),
        pltpu.VMEM((_LB, _D), F32),
        pltpu.VMEM((_LB, 16), F32),
        pltpu.VMEM((16,), F32),
        pltpu.VMEM((8, _D), F32),
        pltpu.VMEM((8, 16), F32),
        pltpu.VMEM_SHARED((_NP, 16), F32),
        pltpu.VMEM_SHARED((_NP, _D), F32),
        pltpu.SemaphoreType.DMA,
        pltpu.SemaphoreType.DMA,
        pltpu.SemaphoreType.DMA,
    ],
)(_sc_edge_body)


def _tc_mid(o0_ref, o1_ref, d0_ref, d1_ref, b1_ref, w2_ref, as2_ref, ad2_ref,
            h2_ref, ats2_ref, atd2_ref, ms2_ref, md2_ref):
    d = d0_ref[...] + d1_ref[...] + 1e-16
    o = o0_ref[...] + o1_ref[...]
    parts = [o[:, 16 * hh:16 * (hh + 1)] / d[:, hh:hh + 1] for hh in range(_H)]
    hcat = jnp.concatenate(parts, axis=1) + b1_ref[...]
    hrelu = jnp.maximum(hcat, 0.0)
    h2 = jnp.dot(hrelu, w2_ref[...], preferred_element_type=F32)
    h2_ref[...] = h2
    a_s = jnp.dot(h2, as2_ref[...], preferred_element_type=F32)
    a_d = jnp.dot(h2, ad2_ref[...], preferred_element_type=F32)
    ats2_ref[...] = a_s
    atd2_ref[...] = a_d
    bs = jnp.max(a_s, axis=0, keepdims=True)
    bd = jnp.max(a_d, axis=0, keepdims=True)
    i = pl.program_id(0)

    @pl.when(i == 0)
    def _():
        ms2_ref[...] = bs
        md2_ref[...] = bd

    @pl.when(i > 0)
    def _():
        ms2_ref[...] = jnp.maximum(ms2_ref[...], bs)
        md2_ref[...] = jnp.maximum(md2_ref[...], bd)


def _mid_call(o0, o1, d0, d1, b1r, W2, As2, Ad2):
    return pl.pallas_call(
        _tc_mid,
        grid=(_N // _BN,),
        in_specs=[
            pl.BlockSpec((_BN, _D), lambda i: (i, 0)),
            pl.BlockSpec((_BN, _D), lambda i: (i, 0)),
            pl.BlockSpec((_BN, 16), lambda i: (i, 0)),
            pl.BlockSpec((_BN, 16), lambda i: (i, 0)),
            pl.BlockSpec((1, _D), lambda i: (0, 0)),
            pl.BlockSpec((_D, _D), lambda i: (0, 0)),
            pl.BlockSpec((_D, 16), lambda i: (0, 0)),
            pl.BlockSpec((_D, 16), lambda i: (0, 0)),
        ],
        out_specs=[
            pl.BlockSpec((_BN, _D), lambda i: (i, 0)),
            pl.BlockSpec((_BN, 16), lambda i: (i, 0)),
            pl.BlockSpec((_BN, 16), lambda i: (i, 0)),
            pl.BlockSpec((1, 16), lambda i: (0, 0)),
            pl.BlockSpec((1, 16), lambda i: (0, 0)),
        ],
        out_shape=[
            jax.ShapeDtypeStruct((_N, _D), F32),
            jax.ShapeDtypeStruct((_N, 16), F32),
            jax.ShapeDtypeStruct((_N, 16), F32),
            jax.ShapeDtypeStruct((1, 16), F32),
            jax.ShapeDtypeStruct((1, 16), F32),
        ],
    )(o0, o1, d0, d1, b1r, W2, As2, Ad2)


def _tc_head(o0_ref, o1_ref, d0_ref, d1_ref, b2_ref, wc1_ref, bc1_ref,
             wc2_ref, bc2_ref, emb_ref, lg_ref):
    d = d0_ref[:, 0:1] + d1_ref[:, 0:1] + 1e-16
    emb = (o0_ref[...] + o1_ref[...]) / d + b2_ref[...]
    emb_ref[...] = emb
    hc = jnp.maximum(
        jnp.dot(emb, wc1_ref[...], preferred_element_type=F32) + bc1_ref[...],
        0.0)
    lg_ref[...] = jnp.dot(hc, wc2_ref[...],
                          preferred_element_type=F32) + bc2_ref[...]


def _head_call(o0, o1, d0, d1, b2r, Wc1, bc1r, Wc2p, bc2p):
    return pl.pallas_call(
        _tc_head,
        grid=(_N // _BN,),
        in_specs=[
            pl.BlockSpec((_BN, _D), lambda i: (i, 0)),
            pl.BlockSpec((_BN, _D), lambda i: (i, 0)),
            pl.BlockSpec((_BN, 16), lambda i: (i, 0)),
            pl.BlockSpec((_BN, 16), lambda i: (i, 0)),
            pl.BlockSpec((1, _D), lambda i: (0, 0)),
            pl.BlockSpec((_D, _D), lambda i: (0, 0)),
            pl.BlockSpec((1, _D), lambda i: (0, 0)),
            pl.BlockSpec((_D, _D), lambda i: (0, 0)),
            pl.BlockSpec((1, _D), lambda i: (0, 0)),
        ],
        out_specs=[
            pl.BlockSpec((_BN, _D), lambda i: (i, 0)),
            pl.BlockSpec((_BN, _D), lambda i: (i, 0)),
        ],
        out_shape=[
            jax.ShapeDtypeStruct((_N, _D), F32),
            jax.ShapeDtypeStruct((_N, _D), F32),
        ],
    )(o0, o1, d0, d1, b2r, Wc1, bc1r, Wc2p, bc2p)


def kernel(x, edge_index, W1, att_src1, att_dst1, b1, W2, att_src2, att_dst2,
           b2, Wc1, bc1, Wc2, bc2):
    src = edge_index[0].astype(jnp.int32)
    dst = edge_index[1].astype(jnp.int32)

    eye = jnp.eye(_H, dtype=F32)
    a1s = att_src1.reshape(_H, 16)
    a1d = att_dst1.reshape(_H, 16)
    pad8 = jnp.zeros((_D, 8), F32)
    As1 = jnp.concatenate(
        [(a1s[:, :, None] * eye[:, None, :]).reshape(_D, _H), pad8], axis=1)
    Ad1 = jnp.concatenate(
        [(a1d[:, :, None] * eye[:, None, :]).reshape(_D, _H), pad8], axis=1)
    As2 = jnp.tile(att_src2.reshape(_D, 1), (1, 16))
    Ad2 = jnp.tile(att_dst2.reshape(_D, 1), (1, 16))

    b1r = b1.reshape(1, _D)
    b2r = b2.reshape(1, _D)
    bc1r = bc1.reshape(1, _D)
    Wc2p = jnp.concatenate([Wc2, jnp.zeros((_D, _D - 2), F32)], axis=1)
    bc2p = jnp.concatenate([bc2, jnp.zeros((_D - 2,), F32)]).reshape(1, _D)

    h1, ats1, atd1, ms1, md1 = _embed_call(x, W1, As1, Ad1)
    dpart1, opart1 = _sc_edge(src, dst, ats1, atd1, h1,
                              ms1.reshape(16), md1.reshape(16))
    h2, ats2, atd2, ms2, md2 = _mid_call(
        opart1[0, :_N], opart1[1, :_N], dpart1[0, :_N], dpart1[1, :_N],
        b1r, W2, As2, Ad2)
    dpart2, opart2 = _sc_edge(src, dst, ats2, atd2, h2,
                              ms2.reshape(16), md2.reshape(16))
    emb, lgp = _head_call(
        opart2[0, :_N], opart2[1, :_N], dpart2[0, :_N], dpart2[1, :_N],
        b2r, Wc1, bc1r, Wc2p, bc2p)
    return emb, lgp[:, :2]


# speedup vs baseline: 1.4900x; 1.4900x over previous
import functools

import jax
import jax.numpy as jnp
from jax import lax
from jax.experimental import pallas as pl
from jax.experimental.pallas import tpu as pltpu
from jax.experimental.pallas import tpu_sc as plsc

F32 = jnp.float32
_N = 10000
_E = 320000
_H = 8
_D = 128
_LB = 64
_NW = 32
_NBLK = _E // _LB
_BN = 2000
_NP = 10112
_RPT = _NP // 16


def _tc_embed(x_ref, w_ref, as_ref, ad_ref, h_ref, ats_ref, atd_ref,
              ms_ref, md_ref):
    h = jnp.dot(x_ref[...], w_ref[...], preferred_element_type=F32)
    h_ref[...] = h
    a_s = jnp.dot(h, as_ref[...], preferred_element_type=F32)
    a_d = jnp.dot(h, ad_ref[...], preferred_element_type=F32)
    ats_ref[...] = a_s
    atd_ref[...] = a_d
    bs = jnp.max(a_s, axis=0, keepdims=True)
    bd = jnp.max(a_d, axis=0, keepdims=True)
    i = pl.program_id(0)

    @pl.when(i == 0)
    def _():
        ms_ref[...] = bs
        md_ref[...] = bd

    @pl.when(i > 0)
    def _():
        ms_ref[...] = jnp.maximum(ms_ref[...], bs)
        md_ref[...] = jnp.maximum(md_ref[...], bd)


def _embed_call(x, W, As, Ad):
    return pl.pallas_call(
        _tc_embed,
        grid=(_N // _BN,),
        in_specs=[
            pl.BlockSpec((_BN, _D), lambda i: (i, 0)),
            pl.BlockSpec((_D, _D), lambda i: (0, 0)),
            pl.BlockSpec((_D, 16), lambda i: (0, 0)),
            pl.BlockSpec((_D, 16), lambda i: (0, 0)),
        ],
        out_specs=[
            pl.BlockSpec((_BN, _D), lambda i: (i, 0)),
            pl.BlockSpec((_BN, 16), lambda i: (i, 0)),
            pl.BlockSpec((_BN, 16), lambda i: (i, 0)),
            pl.BlockSpec((1, 16), lambda i: (0, 0)),
            pl.BlockSpec((1, 16), lambda i: (0, 0)),
        ],
        out_shape=[
            jax.ShapeDtypeStruct((_N, _D), F32),
            jax.ShapeDtypeStruct((_N, 16), F32),
            jax.ShapeDtypeStruct((_N, 16), F32),
            jax.ShapeDtypeStruct((1, 16), F32),
            jax.ShapeDtypeStruct((1, 16), F32),
        ],
    )(x, W, As, Ad)


def _sc_edge_body(src_hbm, dst_hbm, ats_hbm, atd_hbm, h_hbm, ms_hbm, md_hbm,
                  dpart, opart,
                  sidx, didx, asr, adr, hr, exr, mv, zb, zbd, dsp, osp,
                  sem_a, sem_b, sem_c):
    cid = lax.axis_index("c")
    sid = lax.axis_index("s")
    wid = sid * 2 + cid

    pltpu.sync_copy(ms_hbm, mv)
    m_s = mv[...]
    pltpu.sync_copy(md_hbm, mv)
    msum = m_s + mv[...]
    mshift = jnp.where(msum > 0, msum, 0.2 * msum)

    z16 = jnp.zeros((16,), F32)
    for r in range(8):
        for k in range(_D // 16):
            zb[r, pl.ds(16 * k, 16)] = z16
        zbd[r, pl.ds(0, 16)] = z16
    rbase = pl.multiple_of(sid * _RPT, 8)

    def zero_blk(r, c0):
        off = pl.multiple_of(rbase + r * 8, 8)
        pltpu.sync_copy(zb, osp.at[pl.ds(off, 8)])
        pltpu.sync_copy(zbd, dsp.at[pl.ds(off, 8)])
        return c0

    lax.fori_loop(0, _RPT // 8, zero_blk, 0)
    plsc.subcore_barrier()

    def issue(jb, s):
        base = (wid + jb * _NW) * _LB
        pltpu.sync_copy(src_hbm.at[pl.ds(base, _LB)], sidx.at[s])
        pltpu.sync_copy(dst_hbm.at[pl.ds(base, _LB)], didx.at[s])
        ca = pltpu.async_copy(ats_hbm.at[sidx.at[s]], asr.at[s], sem_a)
        cb = pltpu.async_copy(atd_hbm.at[didx.at[s]], adr.at[s], sem_b)
        cc = pltpu.async_copy(h_hbm.at[sidx.at[s]], hr.at[s], sem_c)
        return ca, cb, cc

    def run(cds, s):
        ca, cb, cc = cds
        ca.wait()
        cb.wait()
        cc.wait()

        def edge(i, c2):
            al = asr[s, i, :] + adr[s, i, :]
            al = jnp.where(al > 0, al, 0.2 * al)
            e = jnp.exp(al - mshift)
            exr[s, i, :] = e
            for hh in range(_H):
                w_hh = e[hh]
                hr[s, i, pl.ds(16 * hh, 16)] = (
                    hr[s, i, pl.ds(16 * hh, 16)] * w_hh)
            return c2

        lax.fori_loop(0, _LB, edge, 0)
        pltpu.sync_copy(exr.at[s], dsp.at[didx.at[s]], add=True)
        pltpu.sync_copy(hr.at[s], osp.at[didx.at[s]], add=True)

    def blk2(jj, carry):
        c0 = issue(2 * jj, 0)
        c1 = issue(2 * jj + 1, 1)
        run(c0, 0)
        run(c1, 1)
        return carry

    nb = _NBLK // _NW
    lax.fori_loop(0, nb // 2, blk2, 0)

    @pl.when(wid < _NBLK - nb * _NW)
    def _():
        run(issue(nb, 0), 0)

    plsc.subcore_barrier()

    pltpu.sync_copy(dsp.at[pl.ds(rbase, _RPT)],
                    dpart.at[cid, pl.ds(rbase, _RPT)])
    pltpu.sync_copy(osp.at[pl.ds(rbase, _RPT)],
                    opart.at[cid, pl.ds(rbase, _RPT)])


_sc_edge = functools.partial(
    pl.kernel,
    out_type=[
        jax.ShapeDtypeStruct((2, _NP, 16), F32),
        jax.ShapeDtypeStruct((2, _NP, _D), F32),
    ],
    mesh=plsc.VectorSubcoreMesh(core_axis_name="c", subcore_axis_name="s"),
    compiler_params=pltpu.CompilerParams(use_tc_tiling_on_sc=False),
    scratch_types=[
        pltpu.VMEM((2, _LB), jnp.int32),
        pltpu.VMEM((2, _LB), jnp.int32),
        pltpu.VMEM((2, _LB, 16), F32),
        pltpu.VMEM((2, _LB, 16), F32),
        pltpu.VMEM((2, _LB, _D), F32),
        pltpu.VMEM((2, _LB, 16), F32),
        pltpu.VMEM((16,), F32),
        pltpu.VMEM((8, _D), F32),
        pltpu.VMEM((8, 16), F32),
        pltpu.VMEM_SHARED((_NP, 16), F32),
        pltpu.VMEM_SHARED((_NP, _D), F32),
        pltpu.SemaphoreType.DMA,
        pltpu.SemaphoreType.DMA,
        pltpu.SemaphoreType.DMA,
    ],
)(_sc_edge_body)


def _tc_mid(o0_ref, o1_ref, d0_ref, d1_ref, b1_ref, w2_ref, as2_ref, ad2_ref,
            h2_ref, ats2_ref, atd2_ref, ms2_ref, md2_ref):
    d = d0_ref[...] + d1_ref[...] + 1e-16
    o = o0_ref[...] + o1_ref[...]
    parts = [o[:, 16 * hh:16 * (hh + 1)] / d[:, hh:hh + 1] for hh in range(_H)]
    hcat = jnp.concatenate(parts, axis=1) + b1_ref[...]
    hrelu = jnp.maximum(hcat, 0.0)
    h2 = jnp.dot(hrelu, w2_ref[...], preferred_element_type=F32)
    h2_ref[...] = h2
    a_s = jnp.dot(h2, as2_ref[...], preferred_element_type=F32)
    a_d = jnp.dot(h2, ad2_ref[...], preferred_element_type=F32)
    ats2_ref[...] = a_s
    atd2_ref[...] = a_d
    bs = jnp.max(a_s, axis=0, keepdims=True)
    bd = jnp.max(a_d, axis=0, keepdims=True)
    i = pl.program_id(0)

    @pl.when(i == 0)
    def _():
        ms2_ref[...] = bs
        md2_ref[...] = bd

    @pl.when(i > 0)
    def _():
        ms2_ref[...] = jnp.maximum(ms2_ref[...], bs)
        md2_ref[...] = jnp.maximum(md2_ref[...], bd)


def _mid_call(o0, o1, d0, d1, b1r, W2, As2, Ad2):
    return pl.pallas_call(
        _tc_mid,
        grid=(_N // _BN,),
        in_specs=[
            pl.BlockSpec((_BN, _D), lambda i: (i, 0)),
            pl.BlockSpec((_BN, _D), lambda i: (i, 0)),
            pl.BlockSpec((_BN, 16), lambda i: (i, 0)),
            pl.BlockSpec((_BN, 16), lambda i: (i, 0)),
            pl.BlockSpec((1, _D), lambda i: (0, 0)),
            pl.BlockSpec((_D, _D), lambda i: (0, 0)),
            pl.BlockSpec((_D, 16), lambda i: (0, 0)),
            pl.BlockSpec((_D, 16), lambda i: (0, 0)),
        ],
        out_specs=[
            pl.BlockSpec((_BN, _D), lambda i: (i, 0)),
            pl.BlockSpec((_BN, 16), lambda i: (i, 0)),
            pl.BlockSpec((_BN, 16), lambda i: (i, 0)),
            pl.BlockSpec((1, 16), lambda i: (0, 0)),
            pl.BlockSpec((1, 16), lambda i: (0, 0)),
        ],
        out_shape=[
            jax.ShapeDtypeStruct((_N, _D), F32),
            jax.ShapeDtypeStruct((_N, 16), F32),
            jax.ShapeDtypeStruct((_N, 16), F32),
            jax.ShapeDtypeStruct((1, 16), F32),
            jax.ShapeDtypeStruct((1, 16), F32),
        ],
    )(o0, o1, d0, d1, b1r, W2, As2, Ad2)


def _tc_head(o0_ref, o1_ref, d0_ref, d1_ref, b2_ref, wc1_ref, bc1_ref,
             wc2_ref, bc2_ref, emb_ref, lg_ref):
    d = d0_ref[:, 0:1] + d1_ref[:, 0:1] + 1e-16
    emb = (o0_ref[...] + o1_ref[...]) / d + b2_ref[...]
    emb_ref[...] = emb
    hc = jnp.maximum(
        jnp.dot(emb, wc1_ref[...], preferred_element_type=F32) + bc1_ref[...],
        0.0)
    lg_ref[...] = jnp.dot(hc, wc2_ref[...],
                          preferred_element_type=F32) + bc2_ref[...]


def _head_call(o0, o1, d0, d1, b2r, Wc1, bc1r, Wc2p, bc2p):
    return pl.pallas_call(
        _tc_head,
        grid=(_N // _BN,),
        in_specs=[
            pl.BlockSpec((_BN, _D), lambda i: (i, 0)),
            pl.BlockSpec((_BN, _D), lambda i: (i, 0)),
            pl.BlockSpec((_BN, 16), lambda i: (i, 0)),
            pl.BlockSpec((_BN, 16), lambda i: (i, 0)),
            pl.BlockSpec((1, _D), lambda i: (0, 0)),
            pl.BlockSpec((_D, _D), lambda i: (0, 0)),
            pl.BlockSpec((1, _D), lambda i: (0, 0)),
            pl.BlockSpec((_D, _D), lambda i: (0, 0)),
            pl.BlockSpec((1, _D), lambda i: (0, 0)),
        ],
        out_specs=[
            pl.BlockSpec((_BN, _D), lambda i: (i, 0)),
            pl.BlockSpec((_BN, _D), lambda i: (i, 0)),
        ],
        out_shape=[
            jax.ShapeDtypeStruct((_N, _D), F32),
            jax.ShapeDtypeStruct((_N, _D), F32),
        ],
    )(o0, o1, d0, d1, b2r, Wc1, bc1r, Wc2p, bc2p)


def kernel(x, edge_index, W1, att_src1, att_dst1, b1, W2, att_src2, att_dst2,
           b2, Wc1, bc1, Wc2, bc2):
    src = edge_index[0].astype(jnp.int32)
    dst = edge_index[1].astype(jnp.int32)

    eye = jnp.eye(_H, dtype=F32)
    a1s = att_src1.reshape(_H, 16)
    a1d = att_dst1.reshape(_H, 16)
    pad8 = jnp.zeros((_D, 8), F32)
    As1 = jnp.concatenate(
        [(a1s[:, :, None] * eye[:, None, :]).reshape(_D, _H), pad8], axis=1)
    Ad1 = jnp.concatenate(
        [(a1d[:, :, None] * eye[:, None, :]).reshape(_D, _H), pad8], axis=1)
    As2 = jnp.tile(att_src2.reshape(_D, 1), (1, 16))
    Ad2 = jnp.tile(att_dst2.reshape(_D, 1), (1, 16))

    b1r = b1.reshape(1, _D)
    b2r = b2.reshape(1, _D)
    bc1r = bc1.reshape(1, _D)
    Wc2p = jnp.concatenate([Wc2, jnp.zeros((_D, _D - 2), F32)], axis=1)
    bc2p = jnp.concatenate([bc2, jnp.zeros((_D - 2,), F32)]).reshape(1, _D)

    h1, ats1, atd1, ms1, md1 = _embed_call(x, W1, As1, Ad1)
    dpart1, opart1 = _sc_edge(src, dst, ats1, atd1, h1,
                              ms1.reshape(16), md1.reshape(16))
    h2, ats2, atd2, ms2, md2 = _mid_call(
        opart1[0, :_N], opart1[1, :_N], dpart1[0, :_N], dpart1[1, :_N],
        b1r, W2, As2, Ad2)
    dpart2, opart2 = _sc_edge(src, dst, ats2, atd2, h2,
                              ms2.reshape(16), md2.reshape(16))
    emb, lgp = _head_call(
        opart2[0, :_N], opart2[1, :_N], dpart2[0, :_N], dpart2[1, :_N],
        b2r, Wc1, bc1r, Wc2p, bc2p)
    return emb, lgp[:, :2]


# speedup vs baseline: 1.4996x; 1.0064x over previous
import functools

import jax
import jax.numpy as jnp
from jax import lax
from jax.experimental import pallas as pl
from jax.experimental.pallas import tpu as pltpu
from jax.experimental.pallas import tpu_sc as plsc

F32 = jnp.float32
_N = 10000
_E = 320000
_H = 8
_D = 128
_LB = 64
_NW = 32
_NBLK = _E // _LB
_BN = 2000
_NP = 10112
_RPT = _NP // 16


def _tc_embed(x_ref, w_ref, as_ref, ad_ref, h_ref, ats_ref, atd_ref,
              ms_ref, md_ref):
    h = jnp.dot(x_ref[...], w_ref[...], preferred_element_type=F32)
    h_ref[...] = h
    a_s = jnp.dot(h, as_ref[...], preferred_element_type=F32)
    a_d = jnp.dot(h, ad_ref[...], preferred_element_type=F32)
    ats_ref[...] = a_s
    atd_ref[...] = a_d
    bs = jnp.max(a_s, axis=0, keepdims=True)
    bd = jnp.max(a_d, axis=0, keepdims=True)
    i = pl.program_id(0)

    @pl.when(i == 0)
    def _():
        ms_ref[...] = bs
        md_ref[...] = bd

    @pl.when(i > 0)
    def _():
        ms_ref[...] = jnp.maximum(ms_ref[...], bs)
        md_ref[...] = jnp.maximum(md_ref[...], bd)


def _embed_call(x, W, As, Ad):
    return pl.pallas_call(
        _tc_embed,
        grid=(_N // _BN,),
        in_specs=[
            pl.BlockSpec((_BN, _D), lambda i: (i, 0)),
            pl.BlockSpec((_D, _D), lambda i: (0, 0)),
            pl.BlockSpec((_D, 16), lambda i: (0, 0)),
            pl.BlockSpec((_D, 16), lambda i: (0, 0)),
        ],
        out_specs=[
            pl.BlockSpec((_BN, _D), lambda i: (i, 0)),
            pl.BlockSpec((_BN, 16), lambda i: (i, 0)),
            pl.BlockSpec((_BN, 16), lambda i: (i, 0)),
            pl.BlockSpec((1, 16), lambda i: (0, 0)),
            pl.BlockSpec((1, 16), lambda i: (0, 0)),
        ],
        out_shape=[
            jax.ShapeDtypeStruct((_N, _D), F32),
            jax.ShapeDtypeStruct((_N, 16), F32),
            jax.ShapeDtypeStruct((_N, 16), F32),
            jax.ShapeDtypeStruct((1, 16), F32),
            jax.ShapeDtypeStruct((1, 16), F32),
        ],
    )(x, W, As, Ad)


def _sc_edge_body(src_hbm, dst_hbm, ats_hbm, atd_hbm, h_hbm, ms_hbm, md_hbm,
                  zo_hbm, zd_hbm, dpart, opart,
                  sidx, didx, asr, adr, hr, exr, mv, dsp, osp,
                  sem_a, sem_b, sem_c):
    cid = lax.axis_index("c")
    sid = lax.axis_index("s")
    wid = sid * 2 + cid

    pltpu.sync_copy(ms_hbm, mv)
    m_s = mv[...]
    pltpu.sync_copy(md_hbm, mv)
    msum = m_s + mv[...]
    mshift = jnp.where(msum > 0, msum, 0.2 * msum)

    rbase = pl.multiple_of(sid * _RPT, 8)
    za = pltpu.async_copy(zo_hbm, osp.at[pl.ds(rbase, _RPT)], sem_a)
    zc = pltpu.async_copy(zd_hbm, dsp.at[pl.ds(rbase, _RPT)], sem_b)
    za.wait()
    zc.wait()
    plsc.subcore_barrier()

    def issue(jb, s):
        base = (wid + jb * _NW) * _LB
        pltpu.sync_copy(src_hbm.at[pl.ds(base, _LB)], sidx.at[s])
        pltpu.sync_copy(dst_hbm.at[pl.ds(base, _LB)], didx.at[s])
        ca = pltpu.async_copy(ats_hbm.at[sidx.at[s]], asr.at[s], sem_a)
        cb = pltpu.async_copy(atd_hbm.at[didx.at[s]], adr.at[s], sem_b)
        cc = pltpu.async_copy(h_hbm.at[sidx.at[s]], hr.at[s], sem_c)
        return ca, cb, cc

    def run(cds, s):
        ca, cb, cc = cds
        ca.wait()
        cb.wait()
        cc.wait()

        def edge(i, c2):
            al = asr[s, i, :] + adr[s, i, :]
            al = jnp.where(al > 0, al, 0.2 * al)
            e = jnp.exp(al - mshift)
            exr[s, i, :] = e
            for hh in range(_H):
                w_hh = e[hh]
                hr[s, i, pl.ds(16 * hh, 16)] = (
                    hr[s, i, pl.ds(16 * hh, 16)] * w_hh)
            return c2

        lax.fori_loop(0, _LB, edge, 0)
        pltpu.sync_copy(exr.at[s], dsp.at[didx.at[s]], add=True)
        pltpu.sync_copy(hr.at[s], osp.at[didx.at[s]], add=True)

    def blk2(jj, carry):
        c0 = issue(2 * jj, 0)
        c1 = issue(2 * jj + 1, 1)
        run(c0, 0)
        run(c1, 1)
        return carry

    nb = _NBLK // _NW
    lax.fori_loop(0, nb // 2, blk2, 0)

    @pl.when(wid < _NBLK - nb * _NW)
    def _():
        run(issue(nb, 0), 0)

    plsc.subcore_barrier()

    pltpu.sync_copy(dsp.at[pl.ds(rbase, _RPT)],
                    dpart.at[cid, pl.ds(rbase, _RPT)])
    pltpu.sync_copy(osp.at[pl.ds(rbase, _RPT)],
                    opart.at[cid, pl.ds(rbase, _RPT)])


_sc_edge = functools.partial(
    pl.kernel,
    out_type=[
        jax.ShapeDtypeStruct((2, _NP, 16), F32),
        jax.ShapeDtypeStruct((2, _NP, _D), F32),
    ],
    mesh=plsc.VectorSubcoreMesh(core_axis_name="c", subcore_axis_name="s"),
    compiler_params=pltpu.CompilerParams(use_tc_tiling_on_sc=False),
    scratch_types=[
        pltpu.VMEM((2, _LB), jnp.int32),
        pltpu.VMEM((2, _LB), jnp.int32),
        pltpu.VMEM((2, _LB, 16), F32),
        pltpu.VMEM((2, _LB, 16), F32),
        pltpu.VMEM((2, _LB, _D), F32),
        pltpu.VMEM((2, _LB, 16), F32),
        pltpu.VMEM((16,), F32),
        pltpu.VMEM_SHARED((_NP, 16), F32),
        pltpu.VMEM_SHARED((_NP, _D), F32),
        pltpu.SemaphoreType.DMA,
        pltpu.SemaphoreType.DMA,
        pltpu.SemaphoreType.DMA,
    ],
)(_sc_edge_body)


def _tc_mid(o0_ref, o1_ref, d0_ref, d1_ref, b1_ref, w2_ref, as2_ref, ad2_ref,
            h2_ref, ats2_ref, atd2_ref, ms2_ref, md2_ref):
    d = d0_ref[...] + d1_ref[...] + 1e-16
    o = o0_ref[...] + o1_ref[...]
    parts = [o[:, 16 * hh:16 * (hh + 1)] / d[:, hh:hh + 1] for hh in range(_H)]
    hcat = jnp.concatenate(parts, axis=1) + b1_ref[...]
    hrelu = jnp.maximum(hcat, 0.0)
    h2 = jnp.dot(hrelu, w2_ref[...], preferred_element_type=F32)
    h2_ref[...] = h2
    a_s = jnp.dot(h2, as2_ref[...], preferred_element_type=F32)
    a_d = jnp.dot(h2, ad2_ref[...], preferred_element_type=F32)
    ats2_ref[...] = a_s
    atd2_ref[...] = a_d
    bs = jnp.max(a_s, axis=0, keepdims=True)
    bd = jnp.max(a_d, axis=0, keepdims=True)
    i = pl.program_id(0)

    @pl.when(i == 0)
    def _():
        ms2_ref[...] = bs
        md2_ref[...] = bd

    @pl.when(i > 0)
    def _():
        ms2_ref[...] = jnp.maximum(ms2_ref[...], bs)
        md2_ref[...] = jnp.maximum(md2_ref[...], bd)


def _mid_call(o0, o1, d0, d1, b1r, W2, As2, Ad2):
    return pl.pallas_call(
        _tc_mid,
        grid=(_N // _BN,),
        in_specs=[
            pl.BlockSpec((_BN, _D), lambda i: (i, 0)),
            pl.BlockSpec((_BN, _D), lambda i: (i, 0)),
            pl.BlockSpec((_BN, 16), lambda i: (i, 0)),
            pl.BlockSpec((_BN, 16), lambda i: (i, 0)),
            pl.BlockSpec((1, _D), lambda i: (0, 0)),
            pl.BlockSpec((_D, _D), lambda i: (0, 0)),
            pl.BlockSpec((_D, 16), lambda i: (0, 0)),
            pl.BlockSpec((_D, 16), lambda i: (0, 0)),
        ],
        out_specs=[
            pl.BlockSpec((_BN, _D), lambda i: (i, 0)),
            pl.BlockSpec((_BN, 16), lambda i: (i, 0)),
            pl.BlockSpec((_BN, 16), lambda i: (i, 0)),
            pl.BlockSpec((1, 16), lambda i: (0, 0)),
            pl.BlockSpec((1, 16), lambda i: (0, 0)),
        ],
        out_shape=[
            jax.ShapeDtypeStruct((_N, _D), F32),
            jax.ShapeDtypeStruct((_N, 16), F32),
            jax.ShapeDtypeStruct((_N, 16), F32),
            jax.ShapeDtypeStruct((1, 16), F32),
            jax.ShapeDtypeStruct((1, 16), F32),
        ],
    )(o0, o1, d0, d1, b1r, W2, As2, Ad2)


def _tc_head(o0_ref, o1_ref, d0_ref, d1_ref, b2_ref, wc1_ref, bc1_ref,
             wc2_ref, bc2_ref, emb_ref, lg_ref):
    d = d0_ref[:, 0:1] + d1_ref[:, 0:1] + 1e-16
    emb = (o0_ref[...] + o1_ref[...]) / d + b2_ref[...]
    emb_ref[...] = emb
    hc = jnp.maximum(
        jnp.dot(emb, wc1_ref[...], preferred_element_type=F32) + bc1_ref[...],
        0.0)
    lg_ref[...] = jnp.dot(hc, wc2_ref[...],
                          preferred_element_type=F32) + bc2_ref[...]


def _head_call(o0, o1, d0, d1, b2r, Wc1, bc1r, Wc2p, bc2p):
    return pl.pallas_call(
        _tc_head,
        grid=(_N // _BN,),
        in_specs=[
            pl.BlockSpec((_BN, _D), lambda i: (i, 0)),
            pl.BlockSpec((_BN, _D), lambda i: (i, 0)),
            pl.BlockSpec((_BN, 16), lambda i: (i, 0)),
            pl.BlockSpec((_BN, 16), lambda i: (i, 0)),
            pl.BlockSpec((1, _D), lambda i: (0, 0)),
            pl.BlockSpec((_D, _D), lambda i: (0, 0)),
            pl.BlockSpec((1, _D), lambda i: (0, 0)),
            pl.BlockSpec((_D, _D), lambda i: (0, 0)),
            pl.BlockSpec((1, _D), lambda i: (0, 0)),
        ],
        out_specs=[
            pl.BlockSpec((_BN, _D), lambda i: (i, 0)),
            pl.BlockSpec((_BN, _D), lambda i: (i, 0)),
        ],
        out_shape=[
            jax.ShapeDtypeStruct((_N, _D), F32),
            jax.ShapeDtypeStruct((_N, _D), F32),
        ],
    )(o0, o1, d0, d1, b2r, Wc1, bc1r, Wc2p, bc2p)


def kernel(x, edge_index, W1, att_src1, att_dst1, b1, W2, att_src2, att_dst2,
           b2, Wc1, bc1, Wc2, bc2):
    src = edge_index[0].astype(jnp.int32)
    dst = edge_index[1].astype(jnp.int32)

    eye = jnp.eye(_H, dtype=F32)
    a1s = att_src1.reshape(_H, 16)
    a1d = att_dst1.reshape(_H, 16)
    pad8 = jnp.zeros((_D, 8), F32)
    As1 = jnp.concatenate(
        [(a1s[:, :, None] * eye[:, None, :]).reshape(_D, _H), pad8], axis=1)
    Ad1 = jnp.concatenate(
        [(a1d[:, :, None] * eye[:, None, :]).reshape(_D, _H), pad8], axis=1)
    As2 = jnp.tile(att_src2.reshape(_D, 1), (1, 16))
    Ad2 = jnp.tile(att_dst2.reshape(_D, 1), (1, 16))

    b1r = b1.reshape(1, _D)
    b2r = b2.reshape(1, _D)
    bc1r = bc1.reshape(1, _D)
    Wc2p = jnp.concatenate([Wc2, jnp.zeros((_D, _D - 2), F32)], axis=1)
    bc2p = jnp.concatenate([bc2, jnp.zeros((_D - 2,), F32)]).reshape(1, _D)

    zo = jnp.zeros((_RPT, _D), F32)
    zd = jnp.zeros((_RPT, 16), F32)
    h1, ats1, atd1, ms1, md1 = _embed_call(x, W1, As1, Ad1)
    dpart1, opart1 = _sc_edge(src, dst, ats1, atd1, h1,
                              ms1.reshape(16), md1.reshape(16), zo, zd)
    h2, ats2, atd2, ms2, md2 = _mid_call(
        opart1[0, :_N], opart1[1, :_N], dpart1[0, :_N], dpart1[1, :_N],
        b1r, W2, As2, Ad2)
    dpart2, opart2 = _sc_edge(src, dst, ats2, atd2, h2,
                              ms2.reshape(16), md2.reshape(16), zo, zd)
    emb, lgp = _head_call(
        opart2[0, :_N], opart2[1, :_N], dpart2[0, :_N], dpart2[1, :_N],
        b2r, Wc1, bc1r, Wc2p, bc2p)
    return emb, lgp[:, :2]
